# baseline (speedup 1.0000x reference)
import jax
import jax.numpy as jnp
from jax.experimental import pallas as pl
from jax.experimental.pallas import tpu as pltpu

_DN = (((0,), (0,)), ((), ()))


def _mmT(w, x):
    return jax.lax.dot_general(w, x, _DN, preferred_element_type=jnp.float32)


def _a2c_t(xs_ref, xp_ref, wfb_ref, w1_ref, w2_ref, bias_ref,
           pol_ref, crit_ref, im_ref):
    n = xs_ref.shape[1]
    ones = jnp.ones((1, n), jnp.float32)


    f2 = jnp.maximum(_mmT(wfb_ref[0:8, :], xs_ref[...])
                     + _mmT(wfb_ref[8:16, :], xp_ref[...])
                     + _mmT(bias_ref[:, 0:128], ones), 0.0)

    h = jnp.maximum(_mmT(w1_ref[...], f2)
                    + _mmT(bias_ref[:, 128:224], ones), 0.0)

    out9 = (_mmT(w2_ref[:, 0:9], h)
            + _mmT(bias_ref[:, 256:265], ones))
    pol_ref[...] = out9[0:4]
    crit_ref[...] = out9[4:5]
    im_ref[...] = out9[5:9]


def kernel(state, state_prev, wfb, w1, w2, bias):
    B, D = state.shape
    no = 4

    xs = state.T
    xp = state_prev.T

    tb = B
    for cand in (32768, 16384, 8192, 4096, 2048, 1024, 512, 256, 128):
        if B % cand == 0:
            tb = cand
            break

    def full(a):
        return pl.BlockSpec(a.shape, lambda i: (0, 0))

    outs = pl.pallas_call(
        _a2c_t,
        out_shape=[
            jax.ShapeDtypeStruct((no, B), jnp.float32),
            jax.ShapeDtypeStruct((1, B), jnp.float32),
            jax.ShapeDtypeStruct((no, B), jnp.float32),
        ],
        grid=(B // tb,),
        in_specs=[
            pl.BlockSpec((D, tb), lambda i: (0, i)),
            pl.BlockSpec((D, tb), lambda i: (0, i)),
            full(wfb), full(w1), full(w2), full(bias),
        ],
        out_specs=[
            pl.BlockSpec((no, tb), lambda i: (0, i)),
            pl.BlockSpec((1, tb), lambda i: (0, i)),
            pl.BlockSpec((no, tb), lambda i: (0, i)),
        ],
        compiler_params=pltpu.CompilerParams(
            dimension_semantics=("parallel",)),
    )(xs, xp, wfb, w1, w2, bias)

    return outs[0].T, outs[1].T, outs[2].T


# speedup vs baseline: 1.5982x; 1.5982x over previous
import jax
import jax.numpy as jnp
from jax.experimental import pallas as pl
from jax.experimental.pallas import tpu as pltpu

_DN = (((0,), (0,)), ((), ()))


def _mmT(w, x):
    return jax.lax.dot_general(w, x, _DN, preferred_element_type=jnp.float32)


def _a2c_t(xs_ref, xp_ref, wfb_ref, w1_ref, w2_ref, bias_ref,
           pol_ref, crit_ref, im_ref):
    n = xs_ref.shape[1]
    ones = jnp.ones((1, n), jnp.float32)


    x2 = jnp.concatenate([xs_ref[...], ones, xp_ref[...]], axis=0)
    wf2 = jnp.concatenate(
        [wfb_ref[0:8, :], bias_ref[:, 0:128], wfb_ref[8:16, :]], axis=0)
    f2 = jnp.maximum(_mmT(wf2, x2), 0.0)

    h = jnp.maximum(_mmT(w1_ref[...], f2)
                    + _mmT(bias_ref[:, 128:224], ones), 0.0)

    out9 = (_mmT(w2_ref[:, 0:9], h)
            + _mmT(bias_ref[:, 256:265], ones))
    pol_ref[...] = out9[0:4]
    crit_ref[...] = out9[4:5]
    im_ref[...] = out9[5:9]


def kernel(state, state_prev, wfb, w1, w2, bias):
    B, D = state.shape
    no = 4

    xs = state.T
    xp = state_prev.T

    tb = B
    for cand in (32768, 16384, 8192, 4096, 2048, 1024, 512, 256, 128):
        if B % cand == 0:
            tb = cand
            break

    def full(a):
        return pl.BlockSpec(a.shape, lambda i: (0, 0))

    outs = pl.pallas_call(
        _a2c_t,
        out_shape=[
            jax.ShapeDtypeStruct((no, B), jnp.float32),
            jax.ShapeDtypeStruct((1, B), jnp.float32),
            jax.ShapeDtypeStruct((no, B), jnp.float32),
        ],
        grid=(B // tb,),
        in_specs=[
            pl.BlockSpec((D, tb), lambda i: (0, i)),
            pl.BlockSpec((D, tb), lambda i: (0, i)),
            full(wfb), full(w1), full(w2), full(bias),
        ],
        out_specs=[
            pl.BlockSpec((no, tb), lambda i: (0, i)),
            pl.BlockSpec((1, tb), lambda i: (0, i)),
            pl.BlockSpec((no, tb), lambda i: (0, i)),
        ],
        compiler_params=pltpu.CompilerParams(
            dimension_semantics=("parallel",)),
    )(xs, xp, wfb, w1, w2, bias)

    return outs[0].T, outs[1].T, outs[2].T
